# Initial kernel scaffold; baseline (speedup 1.0000x reference)
import jax
import jax.numpy as jnp
from jax.experimental import pallas as pl


def kernel(positions, node_atom, batch, edge_index, atom_table, deg_w1, deg_w2, deg_w3, Wq, Wk, Wv, Wo, Wo_last, We1, We2, gamma, beta, Wh1, Wh2):
    raise NotImplementedError("write your pallas kernel here")


# speedup vs baseline: 5.1736x; 5.1736x over previous
import functools

import jax
import jax.numpy as jnp
import numpy as np
from jax.experimental import pallas as pl

N = 10000
E = 320000
D = 256
D_FEAT = 512
NUM_BASIS = 128
H = 8
HD = 32
L = 6
NG = 200
MAX_RADIUS = 6.0
AVG_DEGREE = 44.46
AVG_NUM_NODES = 50.0

BE = 4000
BN = 2000

_CENTERS = np.linspace(0.0, MAX_RADIUS, NUM_BASIS).astype(np.float32)
_WIDTH = MAX_RADIUS / NUM_BASIS


def _silu(x):
    return x * (1.0 / (1.0 + jnp.exp(-x)))


def _edge_feat_body(dist_ref, w1_ref, w2_ref, w3_ref, we1_ref, we2_ref,
                    msg_ref, gate_ref):
    dist = dist_ref[...]
    step = np.float32(MAX_RADIUS / (NUM_BASIS - 1))
    centers = jax.lax.broadcasted_iota(
        jnp.int32, (1, NUM_BASIS), 1).astype(jnp.float32) * step
    x = (dist - centers) / _WIDTH
    rbf = jnp.exp(-0.5 * x * x)
    rad = _silu(jnp.dot(rbf, w1_ref[...], preferred_element_type=jnp.float32))
    rad = _silu(jnp.dot(rad, w2_ref[...], preferred_element_type=jnp.float32))
    msg_ref[...] = jnp.dot(rad, w3_ref[...], preferred_element_type=jnp.float32)
    act = _silu(jnp.dot(rbf, we1_ref[...], preferred_element_type=jnp.float32))
    gate_ref[...] = jnp.dot(act, we2_ref[...], preferred_element_type=jnp.float32)


def _edge_features(dist, deg_w1, deg_w2, deg_w3, We1cat, We2bd):
    grid = (E // BE,)
    return pl.pallas_call(
        _edge_feat_body,
        grid=grid,
        in_specs=[
            pl.BlockSpec((BE, 1), lambda i: (i, 0)),
            pl.BlockSpec((NUM_BASIS, 64), lambda i: (0, 0)),
            pl.BlockSpec((64, 64), lambda i: (0, 0)),
            pl.BlockSpec((64, D), lambda i: (0, 0)),
            pl.BlockSpec((NUM_BASIS, L * 64), lambda i: (0, 0)),
            pl.BlockSpec((L * 64, L * H), lambda i: (0, 0)),
        ],
        out_specs=[
            pl.BlockSpec((BE, D), lambda i: (i, 0)),
            pl.BlockSpec((BE, L * H), lambda i: (i, 0)),
        ],
        out_shape=[
            jax.ShapeDtypeStruct((E, D), jnp.float32),
            jax.ShapeDtypeStruct((E, L * H), jnp.float32),
        ],
    )(dist, deg_w1, deg_w2, deg_w3, We1cat, We2bd)


def _proj_body(x_ref, w_ref, q_ref, kv_ref):
    qkv = jnp.dot(x_ref[...], w_ref[...], preferred_element_type=jnp.float32)
    q_ref[...] = qkv[:, :D]
    kv_ref[...] = qkv[:, D:]


def _proj(x, Wqkv):
    return pl.pallas_call(
        _proj_body,
        grid=(N // BN,),
        in_specs=[
            pl.BlockSpec((BN, D), lambda i: (i, 0)),
            pl.BlockSpec((D, 3 * D), lambda i: (0, 0)),
        ],
        out_specs=[
            pl.BlockSpec((BN, D), lambda i: (i, 0)),
            pl.BlockSpec((BN, 2 * D), lambda i: (i, 0)),
        ],
        out_shape=[
            jax.ShapeDtypeStruct((N, D), jnp.float32),
            jax.ShapeDtypeStruct((N, 2 * D), jnp.float32),
        ],
    )(x, Wqkv)


def _step_body(x_ref, agg_ref, wo_ref, w_ref, xn_ref, q_ref, kv_ref):
    xn = x_ref[...] + jnp.dot(agg_ref[...], wo_ref[...],
                              preferred_element_type=jnp.float32)
    xn_ref[...] = xn
    qkv = jnp.dot(xn, w_ref[...], preferred_element_type=jnp.float32)
    q_ref[...] = qkv[:, :D]
    kv_ref[...] = qkv[:, D:]


def _step(x, agg, Wo_l, Wqkv):
    return pl.pallas_call(
        _step_body,
        grid=(N // BN,),
        in_specs=[
            pl.BlockSpec((BN, D), lambda i: (i, 0)),
            pl.BlockSpec((BN, D), lambda i: (i, 0)),
            pl.BlockSpec((D, D), lambda i: (0, 0)),
            pl.BlockSpec((D, 3 * D), lambda i: (0, 0)),
        ],
        out_specs=[
            pl.BlockSpec((BN, D), lambda i: (i, 0)),
            pl.BlockSpec((BN, D), lambda i: (i, 0)),
            pl.BlockSpec((BN, 2 * D), lambda i: (i, 0)),
        ],
        out_shape=[
            jax.ShapeDtypeStruct((N, D), jnp.float32),
            jax.ShapeDtypeStruct((N, D), jnp.float32),
            jax.ShapeDtypeStruct((N, 2 * D), jnp.float32),
        ],
    )(x, agg, Wo_l, Wqkv)


def _logits_body(layer, qd_ref, ks_ref, gate_ref, out_ref):
    prod = qd_ref[...] * ks_ref[...]
    cols = []
    for h in range(H):
        cols.append(jnp.sum(prod[:, h * HD:(h + 1) * HD], axis=1,
                            keepdims=True))
    out_ref[...] = (jnp.concatenate(cols, axis=1) / np.sqrt(HD).astype(np.float32)
                    + gate_ref[:, layer * H:(layer + 1) * H])


def _logits(qd, kvs, gates6, layer):
    return pl.pallas_call(
        functools.partial(_logits_body, layer),
        grid=(E // BE,),
        in_specs=[
            pl.BlockSpec((BE, D), lambda i: (i, 0)),
            pl.BlockSpec((BE, D), lambda i: (i, 0)),
            pl.BlockSpec((BE, L * H), lambda i: (i, 0)),
        ],
        out_specs=pl.BlockSpec((BE, H), lambda i: (i, 0)),
        out_shape=jax.ShapeDtypeStruct((E, H), jnp.float32),
    )(qd, kvs, gates6)


def _wv_body(logit_ref, mg_ref, zg_ref, vs_ref, out_ref):
    w = jnp.exp(logit_ref[...] - mg_ref[...]) / (zg_ref[...] + 1e-9)
    vs = vs_ref[...]
    cols = []
    for h in range(H):
        cols.append(vs[:, h * HD:(h + 1) * HD] * w[:, h:h + 1])
    out_ref[...] = jnp.concatenate(cols, axis=1)


def _weighted_v(logits, mg, zg, kvs):
    return pl.pallas_call(
        _wv_body,
        grid=(E // BE,),
        in_specs=[
            pl.BlockSpec((BE, H), lambda i: (i, 0)),
            pl.BlockSpec((BE, H), lambda i: (i, 0)),
            pl.BlockSpec((BE, H), lambda i: (i, 0)),
            pl.BlockSpec((BE, D), lambda i: (i, 1)),
        ],
        out_specs=pl.BlockSpec((BE, D), lambda i: (i, 0)),
        out_shape=jax.ShapeDtypeStruct((E, D), jnp.float32),
    )(logits, mg, zg, kvs)


def _last_body(agg_ref, w_ref, out_ref):
    out_ref[...] = jnp.dot(agg_ref[...], w_ref[...],
                           preferred_element_type=jnp.float32)


def _last_proj(agg, Wo_last):
    return pl.pallas_call(
        _last_body,
        grid=(N // BN,),
        in_specs=[
            pl.BlockSpec((BN, D), lambda i: (i, 0)),
            pl.BlockSpec((D, D_FEAT), lambda i: (0, 0)),
        ],
        out_specs=pl.BlockSpec((BN, D_FEAT), lambda i: (i, 0)),
        out_shape=jax.ShapeDtypeStruct((N, D_FEAT), jnp.float32),
    )(agg, Wo_last)


def _head_body(x_ref, g_ref, b_ref, w1_ref, w2_ref, out_ref):
    x = x_ref[...]
    mu = jnp.mean(x, axis=1, keepdims=True)
    xc = x - mu
    var = jnp.mean(xc * xc, axis=1, keepdims=True)
    x = xc / jnp.sqrt(var + 1e-5) * g_ref[...] + b_ref[...]
    h = _silu(jnp.dot(x, w1_ref[...], preferred_element_type=jnp.float32))
    out_ref[...] = jnp.dot(h, w2_ref[...], preferred_element_type=jnp.float32)


def _head(x, gamma, beta, Wh1, Wh2):
    return pl.pallas_call(
        _head_body,
        grid=(N // BN,),
        in_specs=[
            pl.BlockSpec((BN, D_FEAT), lambda i: (i, 0)),
            pl.BlockSpec((1, D_FEAT), lambda i: (0, 0)),
            pl.BlockSpec((1, D_FEAT), lambda i: (0, 0)),
            pl.BlockSpec((D_FEAT, D_FEAT), lambda i: (0, 0)),
            pl.BlockSpec((D_FEAT, 1), lambda i: (0, 0)),
        ],
        out_specs=pl.BlockSpec((BN, 1), lambda i: (i, 0)),
        out_shape=jax.ShapeDtypeStruct((N, 1), jnp.float32),
    )(x, gamma, beta, Wh1, Wh2)


def kernel(positions, node_atom, batch, edge_index, atom_table, deg_w1,
           deg_w2, deg_w3, Wq, Wk, Wv, Wo, Wo_last, We1, We2, gamma, beta,
           Wh1, Wh2):
    src = edge_index[0]
    dst = edge_index[1]

    We1cat = jnp.transpose(We1, (1, 0, 2)).reshape(NUM_BASIS, L * 64)
    We2bd = jnp.zeros((L * 64, L * H), jnp.float32)
    for l in range(L):
        We2bd = We2bd.at[l * 64:(l + 1) * 64, l * H:(l + 1) * H].set(We2[l])
    Wqkv = jnp.concatenate([Wq, Wk, Wv], axis=2)

    edge_vec = jnp.take(positions, src, axis=0) - jnp.take(positions, dst, axis=0)
    dist = jnp.sqrt(jnp.sum(edge_vec ** 2, axis=-1) + 1e-12)[:, None]

    msg, gates6 = _edge_features(dist, deg_w1, deg_w2, deg_w3, We1cat, We2bd)

    deg = jnp.zeros((N, D), jnp.float32).at[dst].add(msg)
    x = jnp.take(atom_table, node_atom, axis=0) + deg / np.sqrt(AVG_DEGREE).astype(np.float32)

    inv_sqrt_hd = None
    for l in range(L):
        if l == 0:
            qx, kvx = _proj(x, Wqkv[0])
        else:
            x, qx, kvx = _step(x, agg, Wo[l - 1], Wqkv[l])
        qd = jnp.take(qx, dst, axis=0)
        kvs = jnp.take(kvx, src, axis=0)
        logits = _logits(qd, kvs, gates6, l)
        m = jax.ops.segment_max(logits, dst, num_segments=N)
        m = jnp.where(jnp.isfinite(m), m, 0.0)
        mg = jnp.take(m, dst, axis=0)
        w = jnp.exp(logits - mg)
        z = jax.ops.segment_sum(w, dst, num_segments=N)
        zg = jnp.take(z, dst, axis=0)
        wv = _weighted_v(logits, mg, zg, kvs)
        agg = jnp.zeros((N, D), jnp.float32).at[dst].add(wv)

    xf = _last_proj(agg, Wo_last)
    node_energy = _head(xf, gamma[None, :], beta[None, :], Wh1, Wh2)
    energy = jax.ops.segment_sum(node_energy, batch, num_segments=NG) / AVG_NUM_NODES
    return energy


# speedup vs baseline: 7.0729x; 1.3671x over previous
import functools

import jax
import jax.numpy as jnp
import numpy as np
from jax import lax
from jax.experimental import pallas as pl
from jax.experimental.pallas import tpu as pltpu
from jax.experimental.pallas import tpu_sc as plsc

N = 10000
E = 320000
D = 256
D_FEAT = 512
NUM_BASIS = 128
H = 8
HD = 32
L = 6
NG = 200
MAX_RADIUS = 6.0
AVG_DEGREE = 44.46
AVG_NUM_NODES = 50.0

BE = 4000
BN = 2000

_CENTERS = np.linspace(0.0, MAX_RADIUS, NUM_BASIS).astype(np.float32)
_WIDTH = MAX_RADIUS / NUM_BASIS


def _silu(x):
    return x * (1.0 / (1.0 + jnp.exp(-x)))


def _edge_feat_body(dist_ref, w1_ref, w2_ref, w3_ref, we1_ref, we2_ref,
                    msg_ref, gate_ref):
    dist = dist_ref[...]
    step = np.float32(MAX_RADIUS / (NUM_BASIS - 1))
    centers = jax.lax.broadcasted_iota(
        jnp.int32, (1, NUM_BASIS), 1).astype(jnp.float32) * step
    x = (dist - centers) / _WIDTH
    rbf = jnp.exp(-0.5 * x * x)
    rad = _silu(jnp.dot(rbf, w1_ref[...], preferred_element_type=jnp.float32))
    rad = _silu(jnp.dot(rad, w2_ref[...], preferred_element_type=jnp.float32))
    msg_ref[...] = jnp.dot(rad, w3_ref[...], preferred_element_type=jnp.float32)
    act = _silu(jnp.dot(rbf, we1_ref[...], preferred_element_type=jnp.float32))
    gate_ref[...] = jnp.dot(act, we2_ref[...], preferred_element_type=jnp.float32)


def _edge_features(dist, deg_w1, deg_w2, deg_w3, We1cat, We2bd):
    grid = (E // BE,)
    return pl.pallas_call(
        _edge_feat_body,
        grid=grid,
        in_specs=[
            pl.BlockSpec((BE, 1), lambda i: (i, 0)),
            pl.BlockSpec((NUM_BASIS, 64), lambda i: (0, 0)),
            pl.BlockSpec((64, 64), lambda i: (0, 0)),
            pl.BlockSpec((64, D), lambda i: (0, 0)),
            pl.BlockSpec((NUM_BASIS, L * 64), lambda i: (0, 0)),
            pl.BlockSpec((L * 64, L * H), lambda i: (0, 0)),
        ],
        out_specs=[
            pl.BlockSpec((BE, D), lambda i: (i, 0)),
            pl.BlockSpec((BE, L * H), lambda i: (i, 0)),
        ],
        out_shape=[
            jax.ShapeDtypeStruct((E, D), jnp.float32),
            jax.ShapeDtypeStruct((E, L * H), jnp.float32),
        ],
    )(dist, deg_w1, deg_w2, deg_w3, We1cat, We2bd)


_NW = 32
_CH = 80
_ROWS_PER_W = E // _NW


def _sc_gather(table, idx, dcols):
    mesh = plsc.VectorSubcoreMesh(core_axis_name="c", subcore_axis_name="s")

    @functools.partial(
        pl.kernel, mesh=mesh,
        out_type=jax.ShapeDtypeStruct((E, dcols), jnp.float32),
        scratch_types=[
            pltpu.VMEM((_CH,), jnp.int32),
            pltpu.VMEM((_CH, dcols), jnp.float32),
            pltpu.SemaphoreType.DMA,
        ],
    )
    def gather_k(table_hbm, idx_hbm, out_hbm, idx_v, rows_v, sem):
        wid = lax.axis_index("s") * 2 + lax.axis_index("c")
        base = wid * _ROWS_PER_W

        def body(j, carry):
            off = base + j * _CH
            pltpu.sync_copy(idx_hbm.at[pl.ds(off, _CH)], idx_v)
            pltpu.async_copy(table_hbm.at[idx_v], rows_v, sem).wait()
            pltpu.sync_copy(rows_v, out_hbm.at[pl.ds(off, _CH)])
            return carry

        lax.fori_loop(0, _ROWS_PER_W // _CH, body, 0)

    return gather_k(table, idx)


def _proj_body(x_ref, w_ref, q_ref, kv_ref):
    qkv = jnp.dot(x_ref[...], w_ref[...], preferred_element_type=jnp.float32)
    q_ref[...] = qkv[:, :D]
    kv_ref[...] = qkv[:, D:]


def _proj(x, Wqkv):
    return pl.pallas_call(
        _proj_body,
        grid=(N // BN,),
        in_specs=[
            pl.BlockSpec((BN, D), lambda i: (i, 0)),
            pl.BlockSpec((D, 3 * D), lambda i: (0, 0)),
        ],
        out_specs=[
            pl.BlockSpec((BN, D), lambda i: (i, 0)),
            pl.BlockSpec((BN, 2 * D), lambda i: (i, 0)),
        ],
        out_shape=[
            jax.ShapeDtypeStruct((N, D), jnp.float32),
            jax.ShapeDtypeStruct((N, 2 * D), jnp.float32),
        ],
    )(x, Wqkv)


def _norm_agg(agg_raw, z):
    inv = 1.0 / (z + 1e-9)
    cols = []
    for h in range(H):
        cols.append(agg_raw[:, h * HD:(h + 1) * HD] * inv[:, h:h + 1])
    return jnp.concatenate(cols, axis=1)


def _step_body(x_ref, agg_ref, z_ref, wo_ref, w_ref, xn_ref, q_ref, kv_ref):
    agg = _norm_agg(agg_ref[...], z_ref[...])
    xn = x_ref[...] + jnp.dot(agg, wo_ref[...],
                              preferred_element_type=jnp.float32)
    xn_ref[...] = xn
    qkv = jnp.dot(xn, w_ref[...], preferred_element_type=jnp.float32)
    q_ref[...] = qkv[:, :D]
    kv_ref[...] = qkv[:, D:]


def _step(x, agg, z, Wo_l, Wqkv):
    return pl.pallas_call(
        _step_body,
        grid=(N // BN,),
        in_specs=[
            pl.BlockSpec((BN, D), lambda i: (i, 0)),
            pl.BlockSpec((BN, D), lambda i: (i, 0)),
            pl.BlockSpec((BN, H), lambda i: (i, 0)),
            pl.BlockSpec((D, D), lambda i: (0, 0)),
            pl.BlockSpec((D, 3 * D), lambda i: (0, 0)),
        ],
        out_specs=[
            pl.BlockSpec((BN, D), lambda i: (i, 0)),
            pl.BlockSpec((BN, D), lambda i: (i, 0)),
            pl.BlockSpec((BN, 2 * D), lambda i: (i, 0)),
        ],
        out_shape=[
            jax.ShapeDtypeStruct((N, D), jnp.float32),
            jax.ShapeDtypeStruct((N, D), jnp.float32),
            jax.ShapeDtypeStruct((N, 2 * D), jnp.float32),
        ],
    )(x, agg, z, Wo_l, Wqkv)


def _logits_body(layer, qd_ref, ks_ref, gate_ref, out_ref):
    prod = qd_ref[...] * ks_ref[...]
    cols = []
    for h in range(H):
        cols.append(jnp.sum(prod[:, h * HD:(h + 1) * HD], axis=1,
                            keepdims=True))
    out_ref[...] = (jnp.concatenate(cols, axis=1) / np.sqrt(HD).astype(np.float32)
                    + gate_ref[:, layer * H:(layer + 1) * H])


def _logits(qd, kvs, gates6, layer):
    return pl.pallas_call(
        functools.partial(_logits_body, layer),
        grid=(E // BE,),
        in_specs=[
            pl.BlockSpec((BE, D), lambda i: (i, 0)),
            pl.BlockSpec((BE, D), lambda i: (i, 0)),
            pl.BlockSpec((BE, L * H), lambda i: (i, 0)),
        ],
        out_specs=pl.BlockSpec((BE, H), lambda i: (i, 0)),
        out_shape=jax.ShapeDtypeStruct((E, H), jnp.float32),
    )(qd, kvs, gates6)


def _wv_body(logit_ref, mg_ref, vs_ref, w_ref, out_ref):
    w = jnp.exp(logit_ref[...] - mg_ref[...])
    w_ref[...] = w
    vs = vs_ref[...]
    cols = []
    for h in range(H):
        cols.append(vs[:, h * HD:(h + 1) * HD] * w[:, h:h + 1])
    out_ref[...] = jnp.concatenate(cols, axis=1)


def _weighted_v(logits, mg, kvs):
    return pl.pallas_call(
        _wv_body,
        grid=(E // BE,),
        in_specs=[
            pl.BlockSpec((BE, H), lambda i: (i, 0)),
            pl.BlockSpec((BE, H), lambda i: (i, 0)),
            pl.BlockSpec((BE, D), lambda i: (i, 1)),
        ],
        out_specs=[
            pl.BlockSpec((BE, H), lambda i: (i, 0)),
            pl.BlockSpec((BE, D), lambda i: (i, 0)),
        ],
        out_shape=[
            jax.ShapeDtypeStruct((E, H), jnp.float32),
            jax.ShapeDtypeStruct((E, D), jnp.float32),
        ],
    )(logits, mg, kvs)


def _last_body(agg_ref, z_ref, w_ref, out_ref):
    agg = _norm_agg(agg_ref[...], z_ref[...])
    out_ref[...] = jnp.dot(agg, w_ref[...],
                           preferred_element_type=jnp.float32)


def _last_proj(agg, z, Wo_last):
    return pl.pallas_call(
        _last_body,
        grid=(N // BN,),
        in_specs=[
            pl.BlockSpec((BN, D), lambda i: (i, 0)),
            pl.BlockSpec((BN, H), lambda i: (i, 0)),
            pl.BlockSpec((D, D_FEAT), lambda i: (0, 0)),
        ],
        out_specs=pl.BlockSpec((BN, D_FEAT), lambda i: (i, 0)),
        out_shape=jax.ShapeDtypeStruct((N, D_FEAT), jnp.float32),
    )(agg, z, Wo_last)


def _head_body(x_ref, g_ref, b_ref, w1_ref, w2_ref, out_ref):
    x = x_ref[...]
    mu = jnp.mean(x, axis=1, keepdims=True)
    xc = x - mu
    var = jnp.mean(xc * xc, axis=1, keepdims=True)
    x = xc / jnp.sqrt(var + 1e-5) * g_ref[...] + b_ref[...]
    h = _silu(jnp.dot(x, w1_ref[...], preferred_element_type=jnp.float32))
    out_ref[...] = jnp.dot(h, w2_ref[...], preferred_element_type=jnp.float32)


def _head(x, gamma, beta, Wh1, Wh2):
    return pl.pallas_call(
        _head_body,
        grid=(N // BN,),
        in_specs=[
            pl.BlockSpec((BN, D_FEAT), lambda i: (i, 0)),
            pl.BlockSpec((1, D_FEAT), lambda i: (0, 0)),
            pl.BlockSpec((1, D_FEAT), lambda i: (0, 0)),
            pl.BlockSpec((D_FEAT, D_FEAT), lambda i: (0, 0)),
            pl.BlockSpec((D_FEAT, 1), lambda i: (0, 0)),
        ],
        out_specs=pl.BlockSpec((BN, 1), lambda i: (i, 0)),
        out_shape=jax.ShapeDtypeStruct((N, 1), jnp.float32),
    )(x, gamma, beta, Wh1, Wh2)


def kernel(positions, node_atom, batch, edge_index, atom_table, deg_w1,
           deg_w2, deg_w3, Wq, Wk, Wv, Wo, Wo_last, We1, We2, gamma, beta,
           Wh1, Wh2):
    src = edge_index[0]
    dst = edge_index[1]

    We1cat = jnp.transpose(We1, (1, 0, 2)).reshape(NUM_BASIS, L * 64)
    We2bd = jnp.zeros((L * 64, L * H), jnp.float32)
    for l in range(L):
        We2bd = We2bd.at[l * 64:(l + 1) * 64, l * H:(l + 1) * H].set(We2[l])
    Wqkv = jnp.concatenate([Wq, Wk, Wv], axis=2)

    edge_vec = jnp.take(positions, src, axis=0) - jnp.take(positions, dst, axis=0)
    dist = jnp.sqrt(jnp.sum(edge_vec ** 2, axis=-1) + 1e-12)[:, None]

    msg, gates6 = _edge_features(dist, deg_w1, deg_w2, deg_w3, We1cat, We2bd)

    deg = jnp.zeros((N, D), jnp.float32).at[dst].add(msg)
    x = jnp.take(atom_table, node_atom, axis=0) + deg / np.sqrt(AVG_DEGREE).astype(np.float32)

    for l in range(L):
        if l == 0:
            qx, kvx = _proj(x, Wqkv[0])
        else:
            x, qx, kvx = _step(x, agg, z, Wo[l - 1], Wqkv[l])
        qd = _sc_gather(qx, dst, D)
        kvs = _sc_gather(kvx, src, 2 * D)
        logits = _logits(qd, kvs, gates6, l)
        m = jax.ops.segment_max(logits, dst, num_segments=N)
        mg = jnp.take(m, dst, axis=0)
        w, wv = _weighted_v(logits, mg, kvs)
        z = jax.ops.segment_sum(w, dst, num_segments=N)
        agg = jnp.zeros((N, D), jnp.float32).at[dst].add(wv)

    xf = _last_proj(agg, z, Wo_last)
    node_energy = _head(xf, gamma[None, :], beta[None, :], Wh1, Wh2)
    energy = jax.ops.segment_sum(node_energy, batch, num_segments=NG) / AVG_NUM_NODES
    return energy
